# baseline (speedup 1.0000x reference)
from functools import partial

import jax
import jax.numpy as jnp
from jax.experimental import pallas as pl
from jax.experimental.pallas import tpu as pltpu


def _body(t_ref, n_ref, p_ref, r_ref, c_ref, w1b_ref, w2b_ref,
          out_ref, s1_ref, s3_ref, m_ref, *, nsteps, rem, inv_temp):
    k = pl.program_id(0)

    @pl.when(k == 0)
    def _init():
        s1_ref[...] = jnp.zeros_like(s1_ref)
        s3_ref[...] = jnp.zeros_like(s3_ref)
        m_ref[...] = jnp.zeros_like(m_ref)

    t = t_ref[...]

    def _accum(n, r, e_mask=None):
        logits = jax.lax.dot_general(
            t, n, (((1,), (1,)), ((), ())),
            preferred_element_type=jnp.float32) * inv_temp
        e = jnp.exp(logits)
        if e_mask is not None:
            e = jnp.where(e_mask, e, 0.0)
        s1_ref[...] += jnp.sum(e, axis=1, keepdims=True)
        s3_ref[...] += jnp.sum(r, axis=1, keepdims=True)
        m_ref[...] += jax.lax.dot_general(
            r, n, (((1,), (0,)), ((), ())),
            preferred_element_type=jnp.float32)

    @pl.when(k < nsteps - 1)
    def _full_tile():
        _accum(n_ref[...], r_ref[...])

    @pl.when(k == nsteps - 1)
    def _tail_and_epilogue():
        n = n_ref[...]
        r = r_ref[...]
        row_ok = jax.lax.broadcasted_iota(jnp.int32, n.shape, 0) < rem
        lane_ok = jax.lax.broadcasted_iota(jnp.int32, r.shape, 1) < rem
        _accum(jnp.where(row_ok, n, 0.0), jnp.where(lane_ok, r, 0.0),
               e_mask=lane_ok)

        s1 = s1_ref[...]
        s3 = s3_ref[...]
        s2 = jnp.sum(t * m_ref[...], axis=1, keepdims=True) * inv_temp
        loss = (s2 - jnp.log(s1) * s3) / (s3 + 1e-8)

        c = c_ref[...]
        bg = jax.nn.softmax(jax.lax.dot_general(
            t, c, (((1,), (1,)), ((), ())),
            preferred_element_type=jnp.float32), axis=1)
        pg = jax.nn.softmax(jax.lax.dot_general(
            p_ref[...], c, (((1,), (1,)), ((), ())),
            preferred_element_type=jnp.float32), axis=1)
        sv = (bg - pg) ** 2
        b = sv.shape[0]
        ones = jnp.ones((b, 1), jnp.float32)
        h = jax.lax.dot_general(
            jnp.concatenate([sv, ones], axis=1), w1b_ref[...],
            (((1,), (1,)), ((), ())),
            preferred_element_type=jnp.float32)
        h = jnp.maximum(h, 0.0)
        z = jax.lax.dot_general(
            jnp.concatenate([h, ones], axis=1), w2b_ref[...],
            (((1,), (1,)), ((), ())),
            preferred_element_type=jnp.float32)
        piw = jax.nn.softplus(z)
        piw_sum = jnp.sum(piw, axis=0, keepdims=True)
        num = jnp.sum(loss * piw, axis=0, keepdims=True)
        out_ref[...] = -num / (piw_sum + 1e-8)


def kernel(target_emb, neighbor_emb, present_user_emb, rating_mat,
           cluster, W1, b1, W2, b2):
    B, D = target_emb.shape
    K = neighbor_emb.shape[0]
    C = cluster.shape[0]
    KT = 1024
    nsteps = pl.cdiv(K, KT)
    rem = K - (nsteps - 1) * KT

    out = pl.pallas_call(
        partial(_body, nsteps=nsteps, rem=rem, inv_temp=1.0 / 5.0),
        grid=(nsteps,),
        in_specs=[
            pl.BlockSpec((B, D), lambda k: (0, 0)),
            pl.BlockSpec((KT, D), lambda k: (k, 0)),
            pl.BlockSpec((B, D), lambda k: (0, 0)),
            pl.BlockSpec((B, KT), lambda k: (0, k)),
            pl.BlockSpec((C, D), lambda k: (0, 0)),
            pl.BlockSpec((D, C + 1), lambda k: (0, 0)),
            pl.BlockSpec((1, D + 1), lambda k: (0, 0)),
        ],
        out_specs=pl.BlockSpec((1, 1), lambda k: (0, 0)),
        out_shape=jax.ShapeDtypeStruct((1, 1), jnp.float32),
        scratch_shapes=[
            pltpu.VMEM((B, 1), jnp.float32),
            pltpu.VMEM((B, 1), jnp.float32),
            pltpu.VMEM((B, D), jnp.float32),
        ],
        compiler_params=pltpu.CompilerParams(
            dimension_semantics=("arbitrary",)),
    )(target_emb, neighbor_emb, present_user_emb, rating_mat, cluster,
      jnp.concatenate([W1, b1[:, None]], axis=1),
      jnp.concatenate([W2, b2[:, None]], axis=1))
    return out[0, 0]


# speedup vs baseline: 3.1599x; 3.1599x over previous
from functools import partial

import jax
import jax.numpy as jnp
from jax.experimental import pallas as pl
from jax.experimental.pallas import tpu as pltpu


def _body(tT_ref, nT_ref, pT_ref, rT_ref, c_ref, w1bT_ref, w2b_ref,
          out_ref, s1_ref, s3_ref, m_ref, *, nsteps, rem, inv_temp):
    k = pl.program_id(0)

    @pl.when(k == 0)
    def _init():
        s1_ref[...] = jnp.zeros_like(s1_ref)
        s3_ref[...] = jnp.zeros_like(s3_ref)
        m_ref[...] = jnp.zeros_like(m_ref)

    tT = tT_ref[...]

    def _accum(nT, rT, e_mask=None):
        logitsT = jax.lax.dot_general(
            nT, tT, (((0,), (0,)), ((), ())),
            preferred_element_type=jnp.float32) * inv_temp
        e = jnp.exp(logitsT)
        if e_mask is not None:
            e = jnp.where(e_mask, e, 0.0)
        s1_ref[...] += jnp.sum(e, axis=0, keepdims=True)
        s3_ref[...] += jnp.sum(rT, axis=0, keepdims=True)
        m_ref[...] += jax.lax.dot_general(
            nT, rT, (((1,), (0,)), ((), ())),
            preferred_element_type=jnp.float32)

    @pl.when(k < nsteps - 1)
    def _full_tile():
        _accum(nT_ref[...], rT_ref[...])

    @pl.when(k == nsteps - 1)
    def _tail_and_epilogue():
        nT = nT_ref[...]
        rT = rT_ref[...]
        col_ok = jax.lax.broadcasted_iota(jnp.int32, nT.shape, 1) < rem
        row_ok = jax.lax.broadcasted_iota(jnp.int32, rT.shape, 0) < rem
        _accum(jnp.where(col_ok, nT, 0.0), jnp.where(row_ok, rT, 0.0),
               e_mask=row_ok)

        s1 = s1_ref[...]
        s3 = s3_ref[...]
        s2 = jnp.sum(tT * m_ref[...], axis=0, keepdims=True) * inv_temp
        loss = (s2 - jnp.log(s1) * s3) / (s3 + 1e-8)

        c = c_ref[...]

        def _soft(xT):
            gT = jax.lax.dot_general(
                c, xT, (((1,), (0,)), ((), ())),
                preferred_element_type=jnp.float32)
            gT = jnp.exp(gT - jnp.max(gT, axis=0, keepdims=True))
            return gT / jnp.sum(gT, axis=0, keepdims=True)

        svT = (_soft(tT) - _soft(pT_ref[...])) ** 2
        nb = svT.shape[1]
        ones = jnp.ones((1, nb), jnp.float32)
        hT = jax.lax.dot_general(
            w1bT_ref[...], jnp.concatenate([svT, ones], axis=0),
            (((0,), (0,)), ((), ())),
            preferred_element_type=jnp.float32)
        hT = jnp.maximum(hT, 0.0)
        zT = jax.lax.dot_general(
            w2b_ref[...], jnp.concatenate([hT, ones], axis=0),
            (((1,), (0,)), ((), ())),
            preferred_element_type=jnp.float32)
        piw = jax.nn.softplus(zT)
        piw_sum = jnp.sum(piw, axis=1, keepdims=True)
        num = jnp.sum(loss * piw, axis=1, keepdims=True)
        out_ref[...] = -num / (piw_sum + 1e-8)


def kernel(target_emb, neighbor_emb, present_user_emb, rating_mat,
           cluster, W1, b1, W2, b2):
    B, D = target_emb.shape
    K = neighbor_emb.shape[0]
    C = cluster.shape[0]
    KT = 1024
    nsteps = pl.cdiv(K, KT)
    rem = K - (nsteps - 1) * KT

    out = pl.pallas_call(
        partial(_body, nsteps=nsteps, rem=rem, inv_temp=1.0 / 5.0),
        grid=(nsteps,),
        in_specs=[
            pl.BlockSpec((D, B), lambda k: (0, 0)),
            pl.BlockSpec((D, KT), lambda k: (0, k)),
            pl.BlockSpec((D, B), lambda k: (0, 0)),
            pl.BlockSpec((KT, B), lambda k: (k, 0)),
            pl.BlockSpec((C, D), lambda k: (0, 0)),
            pl.BlockSpec((C + 1, D), lambda k: (0, 0)),
            pl.BlockSpec((1, D + 1), lambda k: (0, 0)),
        ],
        out_specs=pl.BlockSpec((1, 1), lambda k: (0, 0)),
        out_shape=jax.ShapeDtypeStruct((1, 1), jnp.float32),
        scratch_shapes=[
            pltpu.VMEM((1, B), jnp.float32),
            pltpu.VMEM((1, B), jnp.float32),
            pltpu.VMEM((D, B), jnp.float32),
        ],
        compiler_params=pltpu.CompilerParams(
            dimension_semantics=("arbitrary",)),
    )(target_emb.T, neighbor_emb.T, present_user_emb.T, rating_mat.T,
      cluster,
      jnp.concatenate([W1.T, b1[None, :]], axis=0),
      jnp.concatenate([W2, b2[:, None]], axis=1))
    return out[0, 0]


# speedup vs baseline: 3.3042x; 1.0457x over previous
from functools import partial

import jax
import jax.numpy as jnp
from jax.experimental import pallas as pl
from jax.experimental.pallas import tpu as pltpu


def _body(tT_ref, nT_ref, pT_ref, rT_ref, c_ref, w1bT_ref, w2b_ref,
          out_ref, s1_ref, m_ref, *, nsteps, rem, inv_temp):
    k = pl.program_id(0)

    @pl.when(k == 0)
    def _init():
        s1_ref[...] = jnp.zeros_like(s1_ref)
        m_ref[...] = jnp.zeros_like(m_ref)

    tT = tT_ref[...]
    log2e = 1.4426950408889634
    tTs = (tT * (inv_temp * log2e)).astype(jnp.bfloat16)

    def _accum(nT, rT, e_mask=None):
        nTb = nT.astype(jnp.bfloat16)
        q = jax.lax.dot_general(
            nTb, tTs, (((0,), (0,)), ((), ())),
            preferred_element_type=jnp.float32)
        e = jnp.exp2(q)
        if e_mask is not None:
            e = jnp.where(e_mask, e, 0.0)
        s1_ref[...] += jnp.sum(e, axis=0, keepdims=True)
        n1 = jnp.concatenate(
            [nTb, jnp.ones((1, nT.shape[1]), jnp.bfloat16)], axis=0)
        m_ref[...] += jax.lax.dot_general(
            n1, rT.astype(jnp.bfloat16), (((1,), (0,)), ((), ())),
            preferred_element_type=jnp.float32)

    @pl.when(k < nsteps - 1)
    def _full_tile():
        _accum(nT_ref[...], rT_ref[...])

    @pl.when(k == nsteps - 1)
    def _tail_and_epilogue():
        nT = nT_ref[...]
        rT = rT_ref[...]
        col_ok = jax.lax.broadcasted_iota(jnp.int32, nT.shape, 1) < rem
        row_ok = jax.lax.broadcasted_iota(jnp.int32, rT.shape, 0) < rem
        _accum(jnp.where(col_ok, nT, 0.0), jnp.where(row_ok, rT, 0.0),
               e_mask=row_ok)

        s1 = s1_ref[...]
        m = m_ref[...]
        nd = tT.shape[0]
        s3 = m[nd:, :]
        s2 = jnp.sum(tT * m[:nd, :], axis=0, keepdims=True) * inv_temp
        loss = (s2 - jnp.log(s1) * s3) / (s3 + 1e-8)

        c = c_ref[...]

        def _soft(xT):
            gT = jax.lax.dot_general(
                c, xT, (((1,), (0,)), ((), ())),
                preferred_element_type=jnp.float32)
            gT = jnp.exp(gT - jnp.max(gT, axis=0, keepdims=True))
            return gT / jnp.sum(gT, axis=0, keepdims=True)

        svT = (_soft(tT) - _soft(pT_ref[...])) ** 2
        nb = svT.shape[1]
        ones = jnp.ones((1, nb), jnp.float32)
        hT = jax.lax.dot_general(
            w1bT_ref[...], jnp.concatenate([svT, ones], axis=0),
            (((0,), (0,)), ((), ())),
            preferred_element_type=jnp.float32)
        hT = jnp.maximum(hT, 0.0)
        zT = jax.lax.dot_general(
            w2b_ref[...], jnp.concatenate([hT, ones], axis=0),
            (((1,), (0,)), ((), ())),
            preferred_element_type=jnp.float32)
        piw = jax.nn.softplus(zT)
        piw_sum = jnp.sum(piw, axis=1, keepdims=True)
        num = jnp.sum(loss * piw, axis=1, keepdims=True)
        out_ref[...] = -num / (piw_sum + 1e-8)


def kernel(target_emb, neighbor_emb, present_user_emb, rating_mat,
           cluster, W1, b1, W2, b2):
    B, D = target_emb.shape
    K = neighbor_emb.shape[0]
    C = cluster.shape[0]
    KT = 1024
    nsteps = pl.cdiv(K, KT)
    rem = K - (nsteps - 1) * KT

    out = pl.pallas_call(
        partial(_body, nsteps=nsteps, rem=rem, inv_temp=1.0 / 5.0),
        grid=(nsteps,),
        in_specs=[
            pl.BlockSpec((D, B), lambda k: (0, 0)),
            pl.BlockSpec((D, KT), lambda k: (0, k)),
            pl.BlockSpec((D, B), lambda k: (0, 0)),
            pl.BlockSpec((KT, B), lambda k: (k, 0)),
            pl.BlockSpec((C, D), lambda k: (0, 0)),
            pl.BlockSpec((C + 1, D), lambda k: (0, 0)),
            pl.BlockSpec((1, D + 1), lambda k: (0, 0)),
        ],
        out_specs=pl.BlockSpec((1, 1), lambda k: (0, 0)),
        out_shape=jax.ShapeDtypeStruct((1, 1), jnp.float32),
        scratch_shapes=[
            pltpu.VMEM((1, B), jnp.float32),
            pltpu.VMEM((D + 1, B), jnp.float32),
        ],
        compiler_params=pltpu.CompilerParams(
            dimension_semantics=("arbitrary",)),
    )(target_emb.T, neighbor_emb.T, present_user_emb.T, rating_mat.T,
      cluster,
      jnp.concatenate([W1.T, b1[None, :]], axis=0),
      jnp.concatenate([W2, b2[:, None]], axis=1))
    return out[0, 0]


# speedup vs baseline: 4.3133x; 1.3054x over previous
from functools import partial

import jax
import jax.numpy as jnp
from jax.experimental import pallas as pl
from jax.experimental.pallas import tpu as pltpu


def _body(tT_ref, nT_ref, pT_ref, rT_ref, c_ref, w1bT_ref, w2b_ref,
          out_ref, s1_ref, m_ref, *, nsteps, rem, inv_temp):
    k = pl.program_id(0)

    @pl.when(k == 0)
    def _init():
        s1_ref[...] = jnp.zeros_like(s1_ref)
        m_ref[...] = jnp.zeros_like(m_ref)

    tT = tT_ref[...]
    log2e = 1.4426950408889634
    tTs = (tT * (inv_temp * log2e)).astype(jnp.bfloat16)

    def _accum(nT, rT, e_mask=None):
        nTb = nT.astype(jnp.bfloat16)
        q = jax.lax.dot_general(
            nTb, tTs, (((0,), (0,)), ((), ())),
            preferred_element_type=jnp.float32)
        e = jnp.exp2(q)
        if e_mask is not None:
            e = jnp.where(e_mask, e, 0.0)
        s1_ref[...] += jnp.sum(e, axis=0, keepdims=True)
        n1 = jnp.concatenate(
            [nTb, jnp.ones((1, nT.shape[1]), jnp.bfloat16)], axis=0)
        m_ref[...] += jax.lax.dot_general(
            n1, rT.astype(jnp.bfloat16), (((1,), (0,)), ((), ())),
            preferred_element_type=jnp.float32)

    @pl.when(k < nsteps - 1)
    def _full_tile():
        _accum(nT_ref[...], rT_ref[...])

    @pl.when(k == nsteps - 1)
    def _tail_and_epilogue():
        nT = nT_ref[...]
        rT = rT_ref[...]
        col_ok = jax.lax.broadcasted_iota(jnp.int32, nT.shape, 1) < rem
        row_ok = jax.lax.broadcasted_iota(jnp.int32, rT.shape, 0) < rem
        _accum(jnp.where(col_ok, nT, 0.0), jnp.where(row_ok, rT, 0.0),
               e_mask=row_ok)

        s1 = s1_ref[...]
        m = m_ref[...]
        nd = tT.shape[0]
        s3 = m[nd:, :]
        s2 = jnp.sum(tT * m[:nd, :], axis=0, keepdims=True) * inv_temp
        loss = (s2 - jnp.log(s1) * s3) / (s3 + 1e-8)

        c = c_ref[...]

        def _soft(xT):
            gT = jax.lax.dot_general(
                c, xT, (((1,), (0,)), ((), ())),
                preferred_element_type=jnp.float32)
            gT = jnp.exp(gT - jnp.max(gT, axis=0, keepdims=True))
            return gT / jnp.sum(gT, axis=0, keepdims=True)

        svT = (_soft(tT) - _soft(pT_ref[...])) ** 2
        nb = svT.shape[1]
        ones = jnp.ones((1, nb), jnp.float32)
        hT = jax.lax.dot_general(
            w1bT_ref[...], jnp.concatenate([svT, ones], axis=0),
            (((0,), (0,)), ((), ())),
            preferred_element_type=jnp.float32)
        hT = jnp.maximum(hT, 0.0)
        zT = jax.lax.dot_general(
            w2b_ref[...], jnp.concatenate([hT, ones], axis=0),
            (((1,), (0,)), ((), ())),
            preferred_element_type=jnp.float32)
        piw = jax.nn.softplus(zT)
        piw_sum = jnp.sum(piw, axis=1, keepdims=True)
        num = jnp.sum(loss * piw, axis=1, keepdims=True)
        out_ref[...] = -num / (piw_sum + 1e-8)


def kernel(target_emb, neighbor_emb, present_user_emb, rating_mat,
           cluster, W1, b1, W2, b2):
    B, D = target_emb.shape
    K = neighbor_emb.shape[0]
    C = cluster.shape[0]
    KT = 4096
    nsteps = pl.cdiv(K, KT)
    rem = K - (nsteps - 1) * KT

    out = pl.pallas_call(
        partial(_body, nsteps=nsteps, rem=rem, inv_temp=1.0 / 5.0),
        grid=(nsteps,),
        in_specs=[
            pl.BlockSpec((D, B), lambda k: (0, 0)),
            pl.BlockSpec((D, KT), lambda k: (0, k)),
            pl.BlockSpec((D, B), lambda k: (0, 0)),
            pl.BlockSpec((KT, B), lambda k: (k, 0)),
            pl.BlockSpec((C, D), lambda k: (0, 0)),
            pl.BlockSpec((C + 1, D), lambda k: (0, 0)),
            pl.BlockSpec((1, D + 1), lambda k: (0, 0)),
        ],
        out_specs=pl.BlockSpec((1, 1), lambda k: (0, 0)),
        out_shape=jax.ShapeDtypeStruct((1, 1), jnp.float32),
        scratch_shapes=[
            pltpu.VMEM((1, B), jnp.float32),
            pltpu.VMEM((D + 1, B), jnp.float32),
        ],
        compiler_params=pltpu.CompilerParams(
            dimension_semantics=("arbitrary",)),
    )(target_emb.T, neighbor_emb.T, present_user_emb.T, rating_mat.T,
      cluster,
      jnp.concatenate([W1.T, b1[None, :]], axis=0),
      jnp.concatenate([W2, b2[:, None]], axis=1))
    return out[0, 0]
